# baseline (speedup 1.0000x reference)
import functools

import jax
import jax.numpy as jnp
from jax import lax
from jax.experimental import pallas as pl
from jax.experimental.pallas import tpu as pltpu
from jax.experimental.pallas import tpu_sc as plsc

N = 4096
F = 512
H = 256
K = 128
NHID = 8
NCLASS = 2
BETA = 100.0
ALPHA = 0.5
GAMMA = 0.5

BN = 256
BG = 256
NEG_BIG = -9e15


def _f32(x):
    return x.astype(jnp.float32)


@functools.cache
def _make_sc_gather():
    nc, ns = 2, 16
    nw = nc * ns
    bpw = N // nw
    mesh = plsc.VectorSubcoreMesh(core_axis_name="c", subcore_axis_name="s")

    @functools.partial(
        pl.kernel,
        mesh=mesh,
        out_type=jax.ShapeDtypeStruct((N, F), jnp.float32),
        scratch_types=[
            pltpu.VMEM((bpw,), jnp.int32),
            pltpu.VMEM((bpw, F), jnp.float32),
            pltpu.SemaphoreType.DMA,
        ],
    )
    def gather_k(table_hbm, idx_hbm, out_hbm, idx_v, rows_v, sem):
        wid = lax.axis_index("s") * nc + lax.axis_index("c")
        base = wid * bpw
        pltpu.sync_copy(idx_hbm.at[pl.ds(base, bpw)], idx_v)
        pltpu.async_copy(table_hbm.at[idx_v], rows_v, sem).wait()
        pltpu.sync_copy(rows_v, out_hbm.at[pl.ds(base, bpw)])

    return gather_k


def _gather_rows(table, idx):
    return _make_sc_gather()(table, idx)


def _pass1_body(adj_ref, x_ref, xp_ref, w1_ref, hcat_ref, dinv_ref, adjc_ref):
    a = adj_ref[...]
    adjc_ref[...] = a.astype(jnp.bfloat16)
    deg = jnp.sum(a, axis=1, keepdims=True)
    dinv = lax.rsqrt(deg)
    w1 = w1_ref[...]
    xw = jnp.dot(x_ref[...], w1, preferred_element_type=jnp.float32)
    xwn = jnp.dot(xp_ref[...], w1, preferred_element_type=jnp.float32)
    hcat_ref[...] = (jnp.concatenate([xw, xwn], axis=1) * dinv).astype(jnp.bfloat16)
    dinv_ref[...] = dinv


def _pass1(adj, x, xp, w1):
    g = N // BN
    return pl.pallas_call(
        _pass1_body,
        grid=(g,),
        in_specs=[
            pl.BlockSpec((BN, N), lambda i: (i, 0)),
            pl.BlockSpec((BN, F), lambda i: (i, 0)),
            pl.BlockSpec((BN, F), lambda i: (i, 0)),
            pl.BlockSpec((F, H), lambda i: (0, 0)),
        ],
        out_specs=[
            pl.BlockSpec((BN, 2 * H), lambda i: (i, 0)),
            pl.BlockSpec((BN, 1), lambda i: (i, 0)),
            pl.BlockSpec((BN, N), lambda i: (i, 0)),
        ],
        out_shape=[
            jax.ShapeDtypeStruct((N, 2 * H), jnp.bfloat16),
            jax.ShapeDtypeStruct((N, 1), jnp.float32),
            jax.ShapeDtypeStruct((N, N), jnp.bfloat16),
        ],
    )(adj, x, xp, w1)


def _pass2_body(adj_ref, h_ref, dinv_ref, w2_ref, out_ref):
    y = jnp.dot(adj_ref[...], h_ref[...], preferred_element_type=jnp.float32)
    h = jnp.maximum(y * dinv_ref[...], 0.0).astype(jnp.bfloat16)
    w2 = w2_ref[...].astype(jnp.bfloat16)
    zp = jnp.dot(h[:, :H], w2, preferred_element_type=jnp.float32)
    zn = jnp.dot(h[:, H:], w2, preferred_element_type=jnp.float32)
    out_ref[...] = (jnp.concatenate([zp, zn], axis=1) * dinv_ref[...]).astype(jnp.bfloat16)


def _pass2(adj, hcat, dinv, w2):
    g = N // BN
    return pl.pallas_call(
        _pass2_body,
        grid=(g,),
        in_specs=[
            pl.BlockSpec((BN, N), lambda i: (i, 0)),
            pl.BlockSpec((N, 2 * H), lambda i: (0, 0)),
            pl.BlockSpec((BN, 1), lambda i: (i, 0)),
            pl.BlockSpec((H, H), lambda i: (0, 0)),
        ],
        out_specs=pl.BlockSpec((BN, 2 * H), lambda i: (i, 0)),
        out_shape=jax.ShapeDtypeStruct((N, 2 * H), jnp.bfloat16),
    )(adj, hcat, dinv, w2)


def _pass3_body(adj_ref, h_ref, dinv_ref, watt_ref, aatt_ref,
                pos_ref, neg_ref, whp_ref, whn_ref,
                f1p_ref, f2p_ref, f1n_ref, f2n_ref):
    y = jnp.dot(adj_ref[...], h_ref[...], preferred_element_type=jnp.float32)
    dinv = dinv_ref[...]
    pos = y[:, :H] * dinv
    neg = y[:, H:] * dinv
    pos_ref[...] = pos
    neg_ref[...] = neg
    watt = watt_ref[...]
    a = aatt_ref[...]
    a1 = a[:NHID, :]
    a2 = a[NHID:, :]
    whp = jnp.dot(pos, watt, preferred_element_type=jnp.float32)
    whn = jnp.dot(neg, watt, preferred_element_type=jnp.float32)
    whp_ref[...] = whp
    whn_ref[...] = whn
    f1p_ref[...] = jnp.dot(whp, a1, preferred_element_type=jnp.float32)
    f2p_ref[...] = jnp.dot(whp, a2, preferred_element_type=jnp.float32)
    f1n_ref[...] = jnp.dot(whn, a1, preferred_element_type=jnp.float32)
    f2n_ref[...] = jnp.dot(whn, a2, preferred_element_type=jnp.float32)


def _pass3(adj, hcat, dinv, watt, aatt):
    g = N // BN
    vec = jax.ShapeDtypeStruct((N, 1), jnp.float32)
    return pl.pallas_call(
        _pass3_body,
        grid=(g,),
        in_specs=[
            pl.BlockSpec((BN, N), lambda i: (i, 0)),
            pl.BlockSpec((N, 2 * H), lambda i: (0, 0)),
            pl.BlockSpec((BN, 1), lambda i: (i, 0)),
            pl.BlockSpec((H, NHID), lambda i: (0, 0)),
            pl.BlockSpec((2 * NHID, 1), lambda i: (0, 0)),
        ],
        out_specs=[
            pl.BlockSpec((BN, H), lambda i: (i, 0)),
            pl.BlockSpec((BN, H), lambda i: (i, 0)),
            pl.BlockSpec((BN, NHID), lambda i: (i, 0)),
            pl.BlockSpec((BN, NHID), lambda i: (i, 0)),
            pl.BlockSpec((BN, 1), lambda i: (i, 0)),
            pl.BlockSpec((BN, 1), lambda i: (i, 0)),
            pl.BlockSpec((BN, 1), lambda i: (i, 0)),
            pl.BlockSpec((BN, 1), lambda i: (i, 0)),
        ],
        out_shape=[
            jax.ShapeDtypeStruct((N, H), jnp.float32),
            jax.ShapeDtypeStruct((N, H), jnp.float32),
            jax.ShapeDtypeStruct((N, NHID), jnp.float32),
            jax.ShapeDtypeStruct((N, NHID), jnp.float32),
            vec, vec, vec, vec,
        ],
    )(adj, hcat, dinv, watt, aatt)


def _masked_att_matmul(adj, f1, f2row, wh):
    score = f1 + f2row
    score = jnp.where(score >= 0.0, score, 0.2 * score)
    mask = adj > 0.0
    score = jnp.where(mask, score, NEG_BIG)
    m = jnp.max(score, axis=1, keepdims=True)
    p = jnp.where(mask, jnp.exp(score - m), 0.0)
    s = jnp.sum(p, axis=1, keepdims=True)
    hp = jnp.dot(p, wh, preferred_element_type=jnp.float32)
    return hp / s


def _elu(x):
    return jnp.where(x > 0.0, x, jnp.exp(x) - 1.0)


def _gat1_body(adj_ref, f1p_ref, f2p_ref, f1n_ref, f2n_ref, whp_ref, whn_ref,
               wout_ref, aout_ref,
               wh2p_ref, wh2n_ref, g1p_ref, g2p_ref, g1n_ref, g2n_ref):
    adj = adj_ref[...]
    xp = _elu(_masked_att_matmul(adj, f1p_ref[...], f2p_ref[...], whp_ref[...]))
    xn = _elu(_masked_att_matmul(adj, f1n_ref[...], f2n_ref[...], whn_ref[...]))
    wout = wout_ref[...]
    a = aout_ref[...]
    a1 = a[:NCLASS, :]
    a2 = a[NCLASS:, :]
    wh2p = jnp.dot(xp, wout, preferred_element_type=jnp.float32)
    wh2n = jnp.dot(xn, wout, preferred_element_type=jnp.float32)
    wh2p_ref[...] = wh2p
    wh2n_ref[...] = wh2n
    g1p_ref[...] = jnp.dot(wh2p, a1, preferred_element_type=jnp.float32)
    g2p_ref[...] = jnp.dot(wh2p, a2, preferred_element_type=jnp.float32)
    g1n_ref[...] = jnp.dot(wh2n, a1, preferred_element_type=jnp.float32)
    g2n_ref[...] = jnp.dot(wh2n, a2, preferred_element_type=jnp.float32)


def _gat1(adj, f1p, f2p_row, f1n, f2n_row, whp, whn, wout, aout):
    g = N // BG
    vec = jax.ShapeDtypeStruct((N, 1), jnp.float32)
    blk = lambda r, c: pl.BlockSpec((r, c), lambda i: (i, 0))
    full = lambda r, c: pl.BlockSpec((r, c), lambda i: (0, 0))
    return pl.pallas_call(
        _gat1_body,
        grid=(g,),
        in_specs=[
            blk(BG, N),
            blk(BG, 1), full(1, N),
            blk(BG, 1), full(1, N),
            full(N, NHID), full(N, NHID),
            full(NHID, NCLASS), full(2 * NCLASS, 1),
        ],
        out_specs=[
            blk(BG, NCLASS), blk(BG, NCLASS),
            blk(BG, 1), blk(BG, 1), blk(BG, 1), blk(BG, 1),
        ],
        out_shape=[
            jax.ShapeDtypeStruct((N, NCLASS), jnp.float32),
            jax.ShapeDtypeStruct((N, NCLASS), jnp.float32),
            vec, vec, vec, vec,
        ],
    )(adj, f1p, f2p_row, f1n, f2n_row, whp, whn, wout, aout)


def _gat2_body(adj_ref, g1p_ref, g2p_ref, g1n_ref, g2n_ref, wh2p_ref, wh2n_ref,
               attp_ref, attn_ref):
    adj = adj_ref[...]
    attp_ref[...] = _elu(
        _masked_att_matmul(adj, g1p_ref[...], g2p_ref[...], wh2p_ref[...]))
    attn_ref[...] = _elu(
        _masked_att_matmul(adj, g1n_ref[...], g2n_ref[...], wh2n_ref[...]))


def _gat2(adj, g1p, g2p_row, g1n, g2n_row, wh2p, wh2n):
    g = N // BG
    blk = lambda r, c: pl.BlockSpec((r, c), lambda i: (i, 0))
    full = lambda r, c: pl.BlockSpec((r, c), lambda i: (0, 0))
    return pl.pallas_call(
        _gat2_body,
        grid=(g,),
        in_specs=[
            blk(BG, N),
            blk(BG, 1), full(1, N),
            blk(BG, 1), full(1, N),
            full(N, NCLASS), full(N, NCLASS),
        ],
        out_specs=[blk(BG, NCLASS), blk(BG, NCLASS)],
        out_shape=[
            jax.ShapeDtypeStruct((N, NCLASS), jnp.float32),
            jax.ShapeDtypeStruct((N, NCLASS), jnp.float32),
        ],
    )(adj, g1p, g2p_row, g1n, g2n_row, wh2p, wh2n)


def _cluster_body(pos_ref, mu_ref, mu_out_ref, r_out_ref, colmean_ref):
    pos = pos_ref[...]
    nrm = jnp.sqrt(jnp.sum(pos * pos, axis=1, keepdims=True))
    data = pos / (nrm + 1e-8)

    def norm_rows(m):
        return m / jnp.sqrt(jnp.sum(m * m, axis=1, keepdims=True))

    ones_col = jnp.ones((N, 1), dtype=jnp.float32)

    def step(carry):
        mu, _ = carry
        mun = norm_rows(mu)
        dist = lax.dot_general(data, mun, (((1,), (1,)), ((), ())),
                               preferred_element_type=jnp.float32)
        z = BETA * dist
        z = z - jnp.max(z, axis=1, keepdims=True)
        e = jnp.exp(z)
        r = e / jnp.sum(e, axis=1, keepdims=True)
        cm = lax.dot_general(r, data, (((0,), (0,)), ((), ())),
                             preferred_element_type=jnp.float32)
        cr = lax.dot_general(r, ones_col, (((0,), (0,)), ((), ())),
                             preferred_element_type=jnp.float32)
        return cm / cr, dist

    mu0 = mu_ref[...]
    mu, dist = lax.fori_loop(0, 11, lambda t, c: step(c),
                             (mu0, jnp.zeros((N, K), dtype=jnp.float32)))
    z = BETA * dist
    z = z - jnp.max(z, axis=1, keepdims=True)
    e = jnp.exp(z)
    r = e / jnp.sum(e, axis=1, keepdims=True)
    mu_out_ref[...] = mu
    r_out_ref[...] = r
    colmean_ref[...] = jnp.mean(pos, axis=0, keepdims=True)


def _cluster(pos, mu_init):
    return pl.pallas_call(
        _cluster_body,
        out_shape=[
            jax.ShapeDtypeStruct((K, H), jnp.float32),
            jax.ShapeDtypeStruct((N, K), jnp.float32),
            jax.ShapeDtypeStruct((1, H), jnp.float32),
        ],
    )(pos, mu_init)


def _bce_sum(x, z):
    return jnp.sum(jnp.maximum(x, 0.0) - x * z + jnp.log(1.0 + jnp.exp(-jnp.abs(x))))


def _finalize_body(pos_ref, neg_ref, r_ref, mu_ref, wdisc_ref, colmean_ref,
                   attp_ref, attn_ref, out_ref):
    i = pl.program_id(0)

    pos = pos_ref[...]
    neg = neg_ref[...]
    gs = 1.0 / (1.0 + jnp.exp(-colmean_ref[...]))
    v = lax.dot_general(wdisc_ref[...], gs, (((1,), (1,)), ((), ())),
                        preferred_element_type=jnp.float32)
    pos_graph = jnp.dot(pos, v, preferred_element_type=jnp.float32)
    neg_graph = jnp.dot(neg, v, preferred_element_type=jnp.float32)
    cs_logit = jnp.dot(r_ref[...], mu_ref[...], preferred_element_type=jnp.float32)
    cs = 1.0 / (1.0 + jnp.exp(-cs_logit))
    pos_cluster = jnp.sum(pos * cs, axis=1, keepdims=True)
    neg_cluster = jnp.sum(neg * cs, axis=1, keepdims=True)

    part = ALPHA * (_bce_sum(pos_graph, 1.0) + _bce_sum(neg_graph, 0.0)) / N
    part += (1.0 - ALPHA) * (_bce_sum(pos_cluster, 1.0) + _bce_sum(neg_cluster, 0.0)) / N
    part += GAMMA * (_bce_sum(attp_ref[...], 1.0) + _bce_sum(attn_ref[...], 0.0)) / (N * NCLASS)

    @pl.when(i == 0)
    def _():
        out_ref[...] = jnp.zeros_like(out_ref)

    out_ref[...] = out_ref[...] + part


def _finalize(pos, neg, r, mu, wdisc, colmean, attp, attn):
    g = N // BN
    blk = lambda r_, c: pl.BlockSpec((r_, c), lambda i: (i, 0))
    full = lambda r_, c: pl.BlockSpec((r_, c), lambda i: (0, 0))
    return pl.pallas_call(
        _finalize_body,
        grid=(g,),
        in_specs=[
            blk(BN, H), blk(BN, H), blk(BN, K),
            full(K, H), full(H, H), full(1, H),
            blk(BN, NCLASS), blk(BN, NCLASS),
        ],
        out_specs=full(1, 1),
        out_shape=jax.ShapeDtypeStruct((1, 1), jnp.float32),
    )(pos, neg, r, mu, wdisc, colmean, attp, attn)


def kernel(features, adj, perm, W_gcn1, W_gcn2, W_disc, W_att, a_att, W_out, a_out, mu_init):
    featp = _gather_rows(features, perm.astype(jnp.int32))
    hcat0, dinv, adj_c = _pass1(adj, features, featp, W_gcn1)
    hcat1 = _pass2(adj_c, hcat0, dinv, W_gcn2)
    (pos, neg, whp, whn, f1p, f2p, f1n, f2n) = _pass3(adj_c, hcat1, dinv, W_att, a_att)
    (wh2p, wh2n, g1p, g2p, g1n, g2n) = _gat1(
        adj_c, f1p, f2p.reshape(1, N), f1n, f2n.reshape(1, N), whp, whn, W_out, a_out)
    attp, attn = _gat2(
        adj_c, g1p, g2p.reshape(1, N), g1n, g2n.reshape(1, N), wh2p, wh2n)
    mu, r, colmean = _cluster(pos, mu_init)
    lmat = _finalize(pos, neg, r, mu, W_disc, colmean, attp, attn)
    return lmat[0, 0]


# speedup vs baseline: 1.0392x; 1.0392x over previous
import functools

import jax
import jax.numpy as jnp
from jax import lax
from jax.experimental import pallas as pl
from jax.experimental.pallas import tpu as pltpu
from jax.experimental.pallas import tpu_sc as plsc

N = 4096
F = 512
H = 256
K = 128
NHID = 8
NCLASS = 2
BETA = 100.0
ALPHA = 0.5
GAMMA = 0.5

BN = 256
BG = 256
NEG_BIG = -9e15


def _f32(x):
    return x.astype(jnp.float32)


@functools.cache
def _make_sc_gather():
    nc, ns = 2, 16
    nw = nc * ns
    bpw = N // nw
    mesh = plsc.VectorSubcoreMesh(core_axis_name="c", subcore_axis_name="s")

    @functools.partial(
        pl.kernel,
        mesh=mesh,
        out_type=jax.ShapeDtypeStruct((N, F), jnp.float32),
        scratch_types=[
            pltpu.VMEM((bpw,), jnp.int32),
            pltpu.VMEM((bpw, F), jnp.float32),
            pltpu.SemaphoreType.DMA,
        ],
    )
    def gather_k(table_hbm, idx_hbm, out_hbm, idx_v, rows_v, sem):
        wid = lax.axis_index("s") * nc + lax.axis_index("c")
        base = wid * bpw
        pltpu.sync_copy(idx_hbm.at[pl.ds(base, bpw)], idx_v)
        pltpu.async_copy(table_hbm.at[idx_v], rows_v, sem).wait()
        pltpu.sync_copy(rows_v, out_hbm.at[pl.ds(base, bpw)])

    return gather_k


def _gather_rows(table, idx):
    return _make_sc_gather()(table, idx)


def _pass1_body(adj_ref, x_ref, xp_ref, w1_ref, hcat_ref, dinv_ref, adjc_ref):
    a = adj_ref[...]
    adjc_ref[...] = a.astype(jnp.bfloat16)
    deg = jnp.sum(a, axis=1, keepdims=True)
    dinv = lax.rsqrt(deg)
    w1 = w1_ref[...]
    xw = jnp.dot(x_ref[...], w1, preferred_element_type=jnp.float32)
    xwn = jnp.dot(xp_ref[...], w1, preferred_element_type=jnp.float32)
    hcat_ref[...] = (jnp.concatenate([xw, xwn], axis=1) * dinv).astype(jnp.bfloat16)
    dinv_ref[...] = dinv


def _pass1(adj, x, xp, w1):
    g = N // BN
    return pl.pallas_call(
        _pass1_body,
        grid=(g,),
        in_specs=[
            pl.BlockSpec((BN, N), lambda i: (i, 0)),
            pl.BlockSpec((BN, F), lambda i: (i, 0)),
            pl.BlockSpec((BN, F), lambda i: (i, 0)),
            pl.BlockSpec((F, H), lambda i: (0, 0)),
        ],
        out_specs=[
            pl.BlockSpec((BN, 2 * H), lambda i: (i, 0)),
            pl.BlockSpec((BN, 1), lambda i: (i, 0)),
            pl.BlockSpec((BN, N), lambda i: (i, 0)),
        ],
        out_shape=[
            jax.ShapeDtypeStruct((N, 2 * H), jnp.bfloat16),
            jax.ShapeDtypeStruct((N, 1), jnp.float32),
            jax.ShapeDtypeStruct((N, N), jnp.bfloat16),
        ],
    )(adj, x, xp, w1)


def _pass2_body(adj_ref, h_ref, dinv_ref, w2_ref, out_ref):
    y = jnp.dot(adj_ref[...], h_ref[...], preferred_element_type=jnp.float32)
    h = jnp.maximum(y * dinv_ref[...], 0.0).astype(jnp.bfloat16)
    w2 = w2_ref[...].astype(jnp.bfloat16)
    zp = jnp.dot(h[:, :H], w2, preferred_element_type=jnp.float32)
    zn = jnp.dot(h[:, H:], w2, preferred_element_type=jnp.float32)
    out_ref[...] = (jnp.concatenate([zp, zn], axis=1) * dinv_ref[...]).astype(jnp.bfloat16)


def _pass2(adj, hcat, dinv, w2):
    g = N // BN
    return pl.pallas_call(
        _pass2_body,
        grid=(g,),
        in_specs=[
            pl.BlockSpec((BN, N), lambda i: (i, 0)),
            pl.BlockSpec((N, 2 * H), lambda i: (0, 0)),
            pl.BlockSpec((BN, 1), lambda i: (i, 0)),
            pl.BlockSpec((H, H), lambda i: (0, 0)),
        ],
        out_specs=pl.BlockSpec((BN, 2 * H), lambda i: (i, 0)),
        out_shape=jax.ShapeDtypeStruct((N, 2 * H), jnp.bfloat16),
    )(adj, hcat, dinv, w2)


def _pass3_body(adj_ref, h_ref, dinv_ref, watt_ref, aatt_ref,
                pos_ref, neg_ref, whp_ref, whn_ref,
                f1p_ref, f2p_ref, f1n_ref, f2n_ref):
    y = jnp.dot(adj_ref[...], h_ref[...], preferred_element_type=jnp.float32)
    dinv = dinv_ref[...]
    pos = y[:, :H] * dinv
    neg = y[:, H:] * dinv
    pos_ref[...] = pos
    neg_ref[...] = neg
    watt = watt_ref[...]
    a = aatt_ref[...]
    a1 = a[:NHID, :]
    a2 = a[NHID:, :]
    whp = jnp.dot(pos, watt, preferred_element_type=jnp.float32)
    whn = jnp.dot(neg, watt, preferred_element_type=jnp.float32)
    whp_ref[...] = whp
    whn_ref[...] = whn
    f1p_ref[...] = jnp.dot(whp, a1, preferred_element_type=jnp.float32)
    f2p_ref[...] = jnp.dot(whp, a2, preferred_element_type=jnp.float32)
    f1n_ref[...] = jnp.dot(whn, a1, preferred_element_type=jnp.float32)
    f2n_ref[...] = jnp.dot(whn, a2, preferred_element_type=jnp.float32)


def _pass3(adj, hcat, dinv, watt, aatt):
    g = N // BN
    vec = jax.ShapeDtypeStruct((N, 1), jnp.float32)
    return pl.pallas_call(
        _pass3_body,
        grid=(g,),
        in_specs=[
            pl.BlockSpec((BN, N), lambda i: (i, 0)),
            pl.BlockSpec((N, 2 * H), lambda i: (0, 0)),
            pl.BlockSpec((BN, 1), lambda i: (i, 0)),
            pl.BlockSpec((H, NHID), lambda i: (0, 0)),
            pl.BlockSpec((2 * NHID, 1), lambda i: (0, 0)),
        ],
        out_specs=[
            pl.BlockSpec((BN, H), lambda i: (i, 0)),
            pl.BlockSpec((BN, H), lambda i: (i, 0)),
            pl.BlockSpec((BN, NHID), lambda i: (i, 0)),
            pl.BlockSpec((BN, NHID), lambda i: (i, 0)),
            pl.BlockSpec((BN, 1), lambda i: (i, 0)),
            pl.BlockSpec((BN, 1), lambda i: (i, 0)),
            pl.BlockSpec((BN, 1), lambda i: (i, 0)),
            pl.BlockSpec((BN, 1), lambda i: (i, 0)),
        ],
        out_shape=[
            jax.ShapeDtypeStruct((N, H), jnp.float32),
            jax.ShapeDtypeStruct((N, H), jnp.float32),
            jax.ShapeDtypeStruct((N, NHID), jnp.float32),
            jax.ShapeDtypeStruct((N, NHID), jnp.float32),
            vec, vec, vec, vec,
        ],
    )(adj, hcat, dinv, watt, aatt)


def _masked_att_matmul(adj, f1, f2row, wh):
    score = f1 + f2row
    score = jnp.where(score >= 0.0, score, 0.2 * score)
    mask = adj > 0.0
    score = jnp.where(mask, score, NEG_BIG)
    m = jnp.max(score, axis=1, keepdims=True)
    p = jnp.exp(score - m)
    s = jnp.sum(p, axis=1, keepdims=True)
    hp = jnp.dot(p.astype(jnp.bfloat16), wh.astype(jnp.bfloat16),
                 preferred_element_type=jnp.float32)
    return hp / s


def _elu(x):
    return jnp.where(x > 0.0, x, jnp.exp(x) - 1.0)


def _gat1_body(adj_ref, f1p_ref, f2p_ref, f1n_ref, f2n_ref, whp_ref, whn_ref,
               wout_ref, aout_ref,
               wh2p_ref, wh2n_ref, g1p_ref, g2p_ref, g1n_ref, g2n_ref):
    adj = adj_ref[...]
    xp = _elu(_masked_att_matmul(adj, f1p_ref[...], f2p_ref[...], whp_ref[...]))
    xn = _elu(_masked_att_matmul(adj, f1n_ref[...], f2n_ref[...], whn_ref[...]))
    wout = wout_ref[...]
    a = aout_ref[...]
    a1 = a[:NCLASS, :]
    a2 = a[NCLASS:, :]
    wh2p = jnp.dot(xp, wout, preferred_element_type=jnp.float32)
    wh2n = jnp.dot(xn, wout, preferred_element_type=jnp.float32)
    wh2p_ref[...] = wh2p
    wh2n_ref[...] = wh2n
    g1p_ref[...] = jnp.dot(wh2p, a1, preferred_element_type=jnp.float32)
    g2p_ref[...] = jnp.dot(wh2p, a2, preferred_element_type=jnp.float32)
    g1n_ref[...] = jnp.dot(wh2n, a1, preferred_element_type=jnp.float32)
    g2n_ref[...] = jnp.dot(wh2n, a2, preferred_element_type=jnp.float32)


def _gat1(adj, f1p, f2p_row, f1n, f2n_row, whp, whn, wout, aout):
    g = N // BG
    vec = jax.ShapeDtypeStruct((N, 1), jnp.float32)
    blk = lambda r, c: pl.BlockSpec((r, c), lambda i: (i, 0))
    full = lambda r, c: pl.BlockSpec((r, c), lambda i: (0, 0))
    return pl.pallas_call(
        _gat1_body,
        grid=(g,),
        in_specs=[
            blk(BG, N),
            blk(BG, 1), full(1, N),
            blk(BG, 1), full(1, N),
            full(N, NHID), full(N, NHID),
            full(NHID, NCLASS), full(2 * NCLASS, 1),
        ],
        out_specs=[
            blk(BG, NCLASS), blk(BG, NCLASS),
            blk(BG, 1), blk(BG, 1), blk(BG, 1), blk(BG, 1),
        ],
        out_shape=[
            jax.ShapeDtypeStruct((N, NCLASS), jnp.float32),
            jax.ShapeDtypeStruct((N, NCLASS), jnp.float32),
            vec, vec, vec, vec,
        ],
    )(adj, f1p, f2p_row, f1n, f2n_row, whp, whn, wout, aout)


def _gat2_body(adj_ref, g1p_ref, g2p_ref, g1n_ref, g2n_ref, wh2p_ref, wh2n_ref,
               attp_ref, attn_ref):
    adj = adj_ref[...]
    attp_ref[...] = _elu(
        _masked_att_matmul(adj, g1p_ref[...], g2p_ref[...], wh2p_ref[...]))
    attn_ref[...] = _elu(
        _masked_att_matmul(adj, g1n_ref[...], g2n_ref[...], wh2n_ref[...]))


def _gat2(adj, g1p, g2p_row, g1n, g2n_row, wh2p, wh2n):
    g = N // BG
    blk = lambda r, c: pl.BlockSpec((r, c), lambda i: (i, 0))
    full = lambda r, c: pl.BlockSpec((r, c), lambda i: (0, 0))
    return pl.pallas_call(
        _gat2_body,
        grid=(g,),
        in_specs=[
            blk(BG, N),
            blk(BG, 1), full(1, N),
            blk(BG, 1), full(1, N),
            full(N, NCLASS), full(N, NCLASS),
        ],
        out_specs=[blk(BG, NCLASS), blk(BG, NCLASS)],
        out_shape=[
            jax.ShapeDtypeStruct((N, NCLASS), jnp.float32),
            jax.ShapeDtypeStruct((N, NCLASS), jnp.float32),
        ],
    )(adj, g1p, g2p_row, g1n, g2n_row, wh2p, wh2n)


def _cluster_body(pos_ref, mu_ref, mu_out_ref, r_out_ref, colmean_ref):
    pos = pos_ref[...]
    nrm = jnp.sqrt(jnp.sum(pos * pos, axis=1, keepdims=True))
    data = pos / (nrm + 1e-8)

    def norm_rows(m):
        return m / jnp.sqrt(jnp.sum(m * m, axis=1, keepdims=True))

    ones_col = jnp.ones((N, 1), dtype=jnp.bfloat16)
    data_bf = data.astype(jnp.bfloat16)

    def step(carry):
        mu, _ = carry
        mun = norm_rows(mu)
        dist = lax.dot_general(data_bf, mun.astype(jnp.bfloat16),
                               (((1,), (1,)), ((), ())),
                               preferred_element_type=jnp.float32)
        z = BETA * dist
        z = z - jnp.max(z, axis=1, keepdims=True)
        e = jnp.exp(z)
        r = (e / jnp.sum(e, axis=1, keepdims=True)).astype(jnp.bfloat16)
        cm = lax.dot_general(r, data_bf, (((0,), (0,)), ((), ())),
                             preferred_element_type=jnp.float32)
        cr = lax.dot_general(r, ones_col, (((0,), (0,)), ((), ())),
                             preferred_element_type=jnp.float32)
        return cm / cr, dist

    mu0 = mu_ref[...]
    mu, dist = lax.fori_loop(0, 11, lambda t, c: step(c),
                             (mu0, jnp.zeros((N, K), dtype=jnp.float32)))
    z = BETA * dist
    z = z - jnp.max(z, axis=1, keepdims=True)
    e = jnp.exp(z)
    r = e / jnp.sum(e, axis=1, keepdims=True)
    mu_out_ref[...] = mu
    r_out_ref[...] = r
    colmean_ref[...] = jnp.mean(pos, axis=0, keepdims=True)


def _cluster(pos, mu_init):
    return pl.pallas_call(
        _cluster_body,
        out_shape=[
            jax.ShapeDtypeStruct((K, H), jnp.float32),
            jax.ShapeDtypeStruct((N, K), jnp.float32),
            jax.ShapeDtypeStruct((1, H), jnp.float32),
        ],
    )(pos, mu_init)


def _bce_sum(x, z):
    return jnp.sum(jnp.maximum(x, 0.0) - x * z + jnp.log(1.0 + jnp.exp(-jnp.abs(x))))


def _finalize_body(pos_ref, neg_ref, r_ref, mu_ref, wdisc_ref, colmean_ref,
                   attp_ref, attn_ref, out_ref):
    i = pl.program_id(0)

    pos = pos_ref[...]
    neg = neg_ref[...]
    gs = 1.0 / (1.0 + jnp.exp(-colmean_ref[...]))
    v = lax.dot_general(wdisc_ref[...], gs, (((1,), (1,)), ((), ())),
                        preferred_element_type=jnp.float32)
    pos_graph = jnp.dot(pos, v, preferred_element_type=jnp.float32)
    neg_graph = jnp.dot(neg, v, preferred_element_type=jnp.float32)
    cs_logit = jnp.dot(r_ref[...].astype(jnp.bfloat16), mu_ref[...].astype(jnp.bfloat16),
                       preferred_element_type=jnp.float32)
    cs = 1.0 / (1.0 + jnp.exp(-cs_logit))
    pos_cluster = jnp.sum(pos * cs, axis=1, keepdims=True)
    neg_cluster = jnp.sum(neg * cs, axis=1, keepdims=True)

    part = ALPHA * (_bce_sum(pos_graph, 1.0) + _bce_sum(neg_graph, 0.0)) / N
    part += (1.0 - ALPHA) * (_bce_sum(pos_cluster, 1.0) + _bce_sum(neg_cluster, 0.0)) / N
    part += GAMMA * (_bce_sum(attp_ref[...], 1.0) + _bce_sum(attn_ref[...], 0.0)) / (N * NCLASS)

    @pl.when(i == 0)
    def _():
        out_ref[...] = jnp.zeros_like(out_ref)

    out_ref[...] = out_ref[...] + part


def _finalize(pos, neg, r, mu, wdisc, colmean, attp, attn):
    g = N // BN
    blk = lambda r_, c: pl.BlockSpec((r_, c), lambda i: (i, 0))
    full = lambda r_, c: pl.BlockSpec((r_, c), lambda i: (0, 0))
    return pl.pallas_call(
        _finalize_body,
        grid=(g,),
        in_specs=[
            blk(BN, H), blk(BN, H), blk(BN, K),
            full(K, H), full(H, H), full(1, H),
            blk(BN, NCLASS), blk(BN, NCLASS),
        ],
        out_specs=full(1, 1),
        out_shape=jax.ShapeDtypeStruct((1, 1), jnp.float32),
    )(pos, neg, r, mu, wdisc, colmean, attp, attn)


def kernel(features, adj, perm, W_gcn1, W_gcn2, W_disc, W_att, a_att, W_out, a_out, mu_init):
    featp = _gather_rows(features, perm.astype(jnp.int32))
    hcat0, dinv, adj_c = _pass1(adj, features, featp, W_gcn1)
    hcat1 = _pass2(adj_c, hcat0, dinv, W_gcn2)
    (pos, neg, whp, whn, f1p, f2p, f1n, f2n) = _pass3(adj_c, hcat1, dinv, W_att, a_att)
    (wh2p, wh2n, g1p, g2p, g1n, g2n) = _gat1(
        adj_c, f1p, f2p.reshape(1, N), f1n, f2n.reshape(1, N), whp, whn, W_out, a_out)
    attp, attn = _gat2(
        adj_c, g1p, g2p.reshape(1, N), g1n, g2n.reshape(1, N), wh2p, wh2n)
    mu, r, colmean = _cluster(pos, mu_init)
    lmat = _finalize(pos, neg, r, mu, W_disc, colmean, attp, attn)
    return lmat[0, 0]


# speedup vs baseline: 1.2239x; 1.1778x over previous
import functools

import jax
import jax.numpy as jnp
from jax import lax
from jax.experimental import pallas as pl
from jax.experimental.pallas import tpu as pltpu
from jax.experimental.pallas import tpu_sc as plsc

N = 4096
F = 512
H = 256
K = 128
NHID = 8
NCLASS = 2
BETA = 100.0
ALPHA = 0.5
GAMMA = 0.5

BN = 256
BG = 256
NEG_BIG = -9e15


def _f32(x):
    return x.astype(jnp.float32)


@functools.cache
def _make_sc_gather():
    nc, ns = 2, 16
    nw = nc * ns
    bpw = N // nw
    mesh = plsc.VectorSubcoreMesh(core_axis_name="c", subcore_axis_name="s")

    @functools.partial(
        pl.kernel,
        mesh=mesh,
        out_type=jax.ShapeDtypeStruct((N, F), jnp.float32),
        scratch_types=[
            pltpu.VMEM((bpw,), jnp.int32),
            pltpu.VMEM((bpw, F), jnp.float32),
            pltpu.SemaphoreType.DMA,
        ],
    )
    def gather_k(table_hbm, idx_hbm, out_hbm, idx_v, rows_v, sem):
        wid = lax.axis_index("s") * nc + lax.axis_index("c")
        base = wid * bpw
        pltpu.sync_copy(idx_hbm.at[pl.ds(base, bpw)], idx_v)
        pltpu.async_copy(table_hbm.at[idx_v], rows_v, sem).wait()
        pltpu.sync_copy(rows_v, out_hbm.at[pl.ds(base, bpw)])

    return gather_k


def _gather_rows(table, idx):
    return _make_sc_gather()(table, idx)


def _pass1_body(adj_ref, x_ref, xp_ref, w1_ref, hcat_ref, dinv_ref, adjc_ref):
    a = adj_ref[...]
    adjc_ref[...] = a.astype(jnp.bfloat16)
    deg = jnp.sum(a, axis=1, keepdims=True)
    dinv = lax.rsqrt(deg)
    w1 = w1_ref[...]
    xw = jnp.dot(x_ref[...], w1, preferred_element_type=jnp.float32)
    xwn = jnp.dot(xp_ref[...], w1, preferred_element_type=jnp.float32)
    hcat_ref[...] = (jnp.concatenate([xw, xwn], axis=1) * dinv).astype(jnp.bfloat16)
    dinv_ref[...] = dinv


def _pass1(adj, x, xp, w1):
    g = N // BN
    return pl.pallas_call(
        _pass1_body,
        grid=(g,),
        in_specs=[
            pl.BlockSpec((BN, N), lambda i: (i, 0)),
            pl.BlockSpec((BN, F), lambda i: (i, 0)),
            pl.BlockSpec((BN, F), lambda i: (i, 0)),
            pl.BlockSpec((F, H), lambda i: (0, 0)),
        ],
        out_specs=[
            pl.BlockSpec((BN, 2 * H), lambda i: (i, 0)),
            pl.BlockSpec((BN, 1), lambda i: (i, 0)),
            pl.BlockSpec((BN, N), lambda i: (i, 0)),
        ],
        out_shape=[
            jax.ShapeDtypeStruct((N, 2 * H), jnp.bfloat16),
            jax.ShapeDtypeStruct((N, 1), jnp.float32),
            jax.ShapeDtypeStruct((N, N), jnp.bfloat16),
        ],
    )(adj, x, xp, w1)


def _pass2_body(adj_ref, h_ref, dinv_ref, w2_ref, out_ref):
    y = jnp.dot(adj_ref[...], h_ref[...], preferred_element_type=jnp.float32)
    h = jnp.maximum(y * dinv_ref[...], 0.0).astype(jnp.bfloat16)
    w2 = w2_ref[...].astype(jnp.bfloat16)
    zp = jnp.dot(h[:, :H], w2, preferred_element_type=jnp.float32)
    zn = jnp.dot(h[:, H:], w2, preferred_element_type=jnp.float32)
    out_ref[...] = (jnp.concatenate([zp, zn], axis=1) * dinv_ref[...]).astype(jnp.bfloat16)


def _pass2(adj, hcat, dinv, w2):
    g = N // BN
    return pl.pallas_call(
        _pass2_body,
        grid=(g,),
        in_specs=[
            pl.BlockSpec((BN, N), lambda i: (i, 0)),
            pl.BlockSpec((N, 2 * H), lambda i: (0, 0)),
            pl.BlockSpec((BN, 1), lambda i: (i, 0)),
            pl.BlockSpec((H, H), lambda i: (0, 0)),
        ],
        out_specs=pl.BlockSpec((BN, 2 * H), lambda i: (i, 0)),
        out_shape=jax.ShapeDtypeStruct((N, 2 * H), jnp.bfloat16),
    )(adj, hcat, dinv, w2)


def _pass3_body(adj_ref, h_ref, dinv_ref, watt_ref, aatt_ref,
                pos_ref, neg_ref, whp_ref, whn_ref,
                f1p_ref, f2p_ref, f1n_ref, f2n_ref):
    y = jnp.dot(adj_ref[...], h_ref[...], preferred_element_type=jnp.float32)
    dinv = dinv_ref[...]
    pos = y[:, :H] * dinv
    neg = y[:, H:] * dinv
    pos_ref[...] = pos
    neg_ref[...] = neg
    watt = watt_ref[...]
    a = aatt_ref[...]
    a1 = a[:NHID, :]
    a2 = a[NHID:, :]
    whp = jnp.dot(pos, watt, preferred_element_type=jnp.float32)
    whn = jnp.dot(neg, watt, preferred_element_type=jnp.float32)
    whp_ref[...] = whp
    whn_ref[...] = whn
    f1p_ref[...] = jnp.dot(whp, a1, preferred_element_type=jnp.float32)
    f2p_ref[...] = jnp.dot(whp, a2, preferred_element_type=jnp.float32)
    f1n_ref[...] = jnp.dot(whn, a1, preferred_element_type=jnp.float32)
    f2n_ref[...] = jnp.dot(whn, a2, preferred_element_type=jnp.float32)


def _pass3(adj, hcat, dinv, watt, aatt):
    g = N // BN
    vec = jax.ShapeDtypeStruct((N, 1), jnp.float32)
    return pl.pallas_call(
        _pass3_body,
        grid=(g,),
        in_specs=[
            pl.BlockSpec((BN, N), lambda i: (i, 0)),
            pl.BlockSpec((N, 2 * H), lambda i: (0, 0)),
            pl.BlockSpec((BN, 1), lambda i: (i, 0)),
            pl.BlockSpec((H, NHID), lambda i: (0, 0)),
            pl.BlockSpec((2 * NHID, 1), lambda i: (0, 0)),
        ],
        out_specs=[
            pl.BlockSpec((BN, H), lambda i: (i, 0)),
            pl.BlockSpec((BN, H), lambda i: (i, 0)),
            pl.BlockSpec((BN, NHID), lambda i: (i, 0)),
            pl.BlockSpec((BN, NHID), lambda i: (i, 0)),
            pl.BlockSpec((BN, 1), lambda i: (i, 0)),
            pl.BlockSpec((BN, 1), lambda i: (i, 0)),
            pl.BlockSpec((BN, 1), lambda i: (i, 0)),
            pl.BlockSpec((BN, 1), lambda i: (i, 0)),
        ],
        out_shape=[
            jax.ShapeDtypeStruct((N, H), jnp.float32),
            jax.ShapeDtypeStruct((N, H), jnp.float32),
            jax.ShapeDtypeStruct((N, NHID), jnp.float32),
            jax.ShapeDtypeStruct((N, NHID), jnp.float32),
            vec, vec, vec, vec,
        ],
    )(adj, hcat, dinv, watt, aatt)


def _masked_att_matmul(adj, f1, f2row, wh_ext):
    f2m = jnp.max(f2row, axis=1, keepdims=True)
    t = f1 + f2m
    m = jnp.where(t >= 0.0, t, 0.2 * t)
    t = f1 + f2row
    score = jnp.where(t >= 0.0, t, 0.2 * t)
    p = adj * jnp.exp(score - m).astype(jnp.bfloat16)
    hp_ext = jnp.dot(p, wh_ext, preferred_element_type=jnp.float32)
    d = hp_ext.shape[1] - 1
    return hp_ext[:, :d] / hp_ext[:, d:]


def _elu(x):
    return jnp.where(x > 0.0, x, jnp.exp(x) - 1.0)


def _gat1_body(adj_ref, f1p_ref, f2p_ref, f1n_ref, f2n_ref, whp_ref, whn_ref,
               wout_ref, aout_ref,
               wh2p_ref, wh2n_ref, g1p_ref, g2p_ref, g1n_ref, g2n_ref):
    adj = adj_ref[...]
    ones = jnp.ones((N, 1), dtype=jnp.float32)
    whpe = jnp.concatenate([whp_ref[...], ones], axis=1).astype(jnp.bfloat16)
    whne = jnp.concatenate([whn_ref[...], ones], axis=1).astype(jnp.bfloat16)
    xp = _elu(_masked_att_matmul(adj, f1p_ref[...], f2p_ref[...], whpe))
    xn = _elu(_masked_att_matmul(adj, f1n_ref[...], f2n_ref[...], whne))
    wout = wout_ref[...]
    a = aout_ref[...]
    a1 = a[:NCLASS, :]
    a2 = a[NCLASS:, :]
    wh2p = jnp.dot(xp, wout, preferred_element_type=jnp.float32)
    wh2n = jnp.dot(xn, wout, preferred_element_type=jnp.float32)
    wh2p_ref[...] = wh2p
    wh2n_ref[...] = wh2n
    g1p_ref[...] = jnp.dot(wh2p, a1, preferred_element_type=jnp.float32)
    g2p_ref[...] = jnp.dot(wh2p, a2, preferred_element_type=jnp.float32)
    g1n_ref[...] = jnp.dot(wh2n, a1, preferred_element_type=jnp.float32)
    g2n_ref[...] = jnp.dot(wh2n, a2, preferred_element_type=jnp.float32)


def _gat1(adj, f1p, f2p_row, f1n, f2n_row, whp, whn, wout, aout):
    g = N // BG
    vec = jax.ShapeDtypeStruct((N, 1), jnp.float32)
    blk = lambda r, c: pl.BlockSpec((r, c), lambda i: (i, 0))
    full = lambda r, c: pl.BlockSpec((r, c), lambda i: (0, 0))
    return pl.pallas_call(
        _gat1_body,
        grid=(g,),
        in_specs=[
            blk(BG, N),
            blk(BG, 1), full(1, N),
            blk(BG, 1), full(1, N),
            full(N, NHID), full(N, NHID),
            full(NHID, NCLASS), full(2 * NCLASS, 1),
        ],
        out_specs=[
            blk(BG, NCLASS), blk(BG, NCLASS),
            blk(BG, 1), blk(BG, 1), blk(BG, 1), blk(BG, 1),
        ],
        out_shape=[
            jax.ShapeDtypeStruct((N, NCLASS), jnp.float32),
            jax.ShapeDtypeStruct((N, NCLASS), jnp.float32),
            vec, vec, vec, vec,
        ],
    )(adj, f1p, f2p_row, f1n, f2n_row, whp, whn, wout, aout)


def _gat2_body(adj_ref, g1p_ref, g2p_ref, g1n_ref, g2n_ref, wh2p_ref, wh2n_ref,
               attp_ref, attn_ref):
    adj = adj_ref[...]
    ones = jnp.ones((N, 1), dtype=jnp.float32)
    wh2pe = jnp.concatenate([wh2p_ref[...], ones], axis=1).astype(jnp.bfloat16)
    wh2ne = jnp.concatenate([wh2n_ref[...], ones], axis=1).astype(jnp.bfloat16)
    attp_ref[...] = _elu(_masked_att_matmul(adj, g1p_ref[...], g2p_ref[...], wh2pe))
    attn_ref[...] = _elu(_masked_att_matmul(adj, g1n_ref[...], g2n_ref[...], wh2ne))


def _gat2(adj, g1p, g2p_row, g1n, g2n_row, wh2p, wh2n):
    g = N // BG
    blk = lambda r, c: pl.BlockSpec((r, c), lambda i: (i, 0))
    full = lambda r, c: pl.BlockSpec((r, c), lambda i: (0, 0))
    return pl.pallas_call(
        _gat2_body,
        grid=(g,),
        in_specs=[
            blk(BG, N),
            blk(BG, 1), full(1, N),
            blk(BG, 1), full(1, N),
            full(N, NCLASS), full(N, NCLASS),
        ],
        out_specs=[blk(BG, NCLASS), blk(BG, NCLASS)],
        out_shape=[
            jax.ShapeDtypeStruct((N, NCLASS), jnp.float32),
            jax.ShapeDtypeStruct((N, NCLASS), jnp.float32),
        ],
    )(adj, g1p, g2p_row, g1n, g2n_row, wh2p, wh2n)


def _cluster_body(pos_ref, mu_ref, mu_out_ref, r_out_ref, colmean_ref):
    pos = pos_ref[...]
    nrm = jnp.sqrt(jnp.sum(pos * pos, axis=1, keepdims=True))
    data = pos / (nrm + 1e-8)

    def norm_rows(m):
        return m / jnp.sqrt(jnp.sum(m * m, axis=1, keepdims=True))

    ones_col = jnp.ones((N, 1), dtype=jnp.bfloat16)
    data_bf = data.astype(jnp.bfloat16)

    def step(carry):
        mu, _ = carry
        mun = norm_rows(mu)
        dist = lax.dot_general(data_bf, mun.astype(jnp.bfloat16),
                               (((1,), (1,)), ((), ())),
                               preferred_element_type=jnp.float32)
        z = BETA * dist
        z = z - jnp.max(z, axis=1, keepdims=True)
        e = jnp.exp(z)
        r = (e / jnp.sum(e, axis=1, keepdims=True)).astype(jnp.bfloat16)
        cm = lax.dot_general(r, data_bf, (((0,), (0,)), ((), ())),
                             preferred_element_type=jnp.float32)
        cr = lax.dot_general(r, ones_col, (((0,), (0,)), ((), ())),
                             preferred_element_type=jnp.float32)
        return cm / cr, dist

    mu0 = mu_ref[...]
    mu, dist = lax.fori_loop(0, 11, lambda t, c: step(c),
                             (mu0, jnp.zeros((N, K), dtype=jnp.float32)))
    z = BETA * dist
    z = z - jnp.max(z, axis=1, keepdims=True)
    e = jnp.exp(z)
    r = e / jnp.sum(e, axis=1, keepdims=True)
    mu_out_ref[...] = mu
    r_out_ref[...] = r
    colmean_ref[...] = jnp.mean(pos, axis=0, keepdims=True)


def _cluster(pos, mu_init):
    return pl.pallas_call(
        _cluster_body,
        out_shape=[
            jax.ShapeDtypeStruct((K, H), jnp.float32),
            jax.ShapeDtypeStruct((N, K), jnp.float32),
            jax.ShapeDtypeStruct((1, H), jnp.float32),
        ],
    )(pos, mu_init)


def _bce_sum(x, z):
    return jnp.sum(jnp.maximum(x, 0.0) - x * z + jnp.log(1.0 + jnp.exp(-jnp.abs(x))))


def _finalize_body(pos_ref, neg_ref, r_ref, mu_ref, wdisc_ref, colmean_ref,
                   attp_ref, attn_ref, out_ref):
    i = pl.program_id(0)

    pos = pos_ref[...]
    neg = neg_ref[...]
    gs = 1.0 / (1.0 + jnp.exp(-colmean_ref[...]))
    v = lax.dot_general(wdisc_ref[...], gs, (((1,), (1,)), ((), ())),
                        preferred_element_type=jnp.float32)
    pos_graph = jnp.dot(pos, v, preferred_element_type=jnp.float32)
    neg_graph = jnp.dot(neg, v, preferred_element_type=jnp.float32)
    cs_logit = jnp.dot(r_ref[...].astype(jnp.bfloat16), mu_ref[...].astype(jnp.bfloat16),
                       preferred_element_type=jnp.float32)
    cs = 1.0 / (1.0 + jnp.exp(-cs_logit))
    pos_cluster = jnp.sum(pos * cs, axis=1, keepdims=True)
    neg_cluster = jnp.sum(neg * cs, axis=1, keepdims=True)

    part = ALPHA * (_bce_sum(pos_graph, 1.0) + _bce_sum(neg_graph, 0.0)) / N
    part += (1.0 - ALPHA) * (_bce_sum(pos_cluster, 1.0) + _bce_sum(neg_cluster, 0.0)) / N
    part += GAMMA * (_bce_sum(attp_ref[...], 1.0) + _bce_sum(attn_ref[...], 0.0)) / (N * NCLASS)

    @pl.when(i == 0)
    def _():
        out_ref[...] = jnp.zeros_like(out_ref)

    out_ref[...] = out_ref[...] + part


def _finalize(pos, neg, r, mu, wdisc, colmean, attp, attn):
    g = N // BN
    blk = lambda r_, c: pl.BlockSpec((r_, c), lambda i: (i, 0))
    full = lambda r_, c: pl.BlockSpec((r_, c), lambda i: (0, 0))
    return pl.pallas_call(
        _finalize_body,
        grid=(g,),
        in_specs=[
            blk(BN, H), blk(BN, H), blk(BN, K),
            full(K, H), full(H, H), full(1, H),
            blk(BN, NCLASS), blk(BN, NCLASS),
        ],
        out_specs=full(1, 1),
        out_shape=jax.ShapeDtypeStruct((1, 1), jnp.float32),
    )(pos, neg, r, mu, wdisc, colmean, attp, attn)


def kernel(features, adj, perm, W_gcn1, W_gcn2, W_disc, W_att, a_att, W_out, a_out, mu_init):
    featp = _gather_rows(features, perm.astype(jnp.int32))
    hcat0, dinv, adj_c = _pass1(adj, features, featp, W_gcn1)
    hcat1 = _pass2(adj_c, hcat0, dinv, W_gcn2)
    (pos, neg, whp, whn, f1p, f2p, f1n, f2n) = _pass3(adj_c, hcat1, dinv, W_att, a_att)
    (wh2p, wh2n, g1p, g2p, g1n, g2n) = _gat1(
        adj_c, f1p, f2p.reshape(1, N), f1n, f2n.reshape(1, N), whp, whn, W_out, a_out)
    attp, attn = _gat2(
        adj_c, g1p, g2p.reshape(1, N), g1n, g2n.reshape(1, N), wh2p, wh2n)
    mu, r, colmean = _cluster(pos, mu_init)
    lmat = _finalize(pos, neg, r, mu, W_disc, colmean, attp, attn)
    return lmat[0, 0]


# speedup vs baseline: 1.2851x; 1.0501x over previous
import functools

import jax
import jax.numpy as jnp
from jax import lax
from jax.experimental import pallas as pl
from jax.experimental.pallas import tpu as pltpu
from jax.experimental.pallas import tpu_sc as plsc

N = 4096
F = 512
H = 256
K = 128
NHID = 8
NCLASS = 2
BETA = 100.0
ALPHA = 0.5
GAMMA = 0.5

BN = 256
BG = 256
NEG_BIG = -9e15


def _f32(x):
    return x.astype(jnp.float32)


@functools.cache
def _make_sc_gather():
    nc, ns = 2, 16
    nw = nc * ns
    bpw = N // nw
    mesh = plsc.VectorSubcoreMesh(core_axis_name="c", subcore_axis_name="s")

    @functools.partial(
        pl.kernel,
        mesh=mesh,
        out_type=jax.ShapeDtypeStruct((N, F), jnp.float32),
        scratch_types=[
            pltpu.VMEM((bpw,), jnp.int32),
            pltpu.VMEM((bpw, F), jnp.float32),
            pltpu.SemaphoreType.DMA,
        ],
    )
    def gather_k(table_hbm, idx_hbm, out_hbm, idx_v, rows_v, sem):
        wid = lax.axis_index("s") * nc + lax.axis_index("c")
        base = wid * bpw
        pltpu.sync_copy(idx_hbm.at[pl.ds(base, bpw)], idx_v)
        pltpu.async_copy(table_hbm.at[idx_v], rows_v, sem).wait()
        pltpu.sync_copy(rows_v, out_hbm.at[pl.ds(base, bpw)])

    return gather_k


def _gather_rows(table, idx):
    return _make_sc_gather()(table, idx)


def _pass1_body(adj_ref, x_ref, xp_ref, w1_ref, hcat_ref, dinv_ref, adjc_ref):
    a = adj_ref[...]
    adjc_ref[...] = a.astype(jnp.bfloat16)
    deg = jnp.sum(a, axis=1, keepdims=True)
    dinv = lax.rsqrt(deg)
    w1 = w1_ref[...]
    xw = jnp.dot(x_ref[...], w1, preferred_element_type=jnp.float32)
    xwn = jnp.dot(xp_ref[...], w1, preferred_element_type=jnp.float32)
    hcat_ref[...] = (jnp.concatenate([xw, xwn], axis=1) * dinv).astype(jnp.bfloat16)
    dinv_ref[...] = dinv


def _pass1(adj, x, xp, w1):
    g = N // BN
    return pl.pallas_call(
        _pass1_body,
        grid=(g,),
        in_specs=[
            pl.BlockSpec((BN, N), lambda i: (i, 0)),
            pl.BlockSpec((BN, F), lambda i: (i, 0)),
            pl.BlockSpec((BN, F), lambda i: (i, 0)),
            pl.BlockSpec((F, H), lambda i: (0, 0)),
        ],
        out_specs=[
            pl.BlockSpec((BN, 2 * H), lambda i: (i, 0)),
            pl.BlockSpec((BN, 1), lambda i: (i, 0)),
            pl.BlockSpec((BN, N), lambda i: (i, 0)),
        ],
        out_shape=[
            jax.ShapeDtypeStruct((N, 2 * H), jnp.bfloat16),
            jax.ShapeDtypeStruct((N, 1), jnp.float32),
            jax.ShapeDtypeStruct((N, N), jnp.bfloat16),
        ],
    )(adj, x, xp, w1)


def _pass2_body(adj_ref, h_ref, dinv_ref, w2_ref, out_ref):
    y = jnp.dot(adj_ref[...], h_ref[...], preferred_element_type=jnp.float32)
    h = jnp.maximum(y * dinv_ref[...], 0.0).astype(jnp.bfloat16)
    w2 = w2_ref[...].astype(jnp.bfloat16)
    zp = jnp.dot(h[:, :H], w2, preferred_element_type=jnp.float32)
    zn = jnp.dot(h[:, H:], w2, preferred_element_type=jnp.float32)
    out_ref[...] = (jnp.concatenate([zp, zn], axis=1) * dinv_ref[...]).astype(jnp.bfloat16)


def _pass2(adj, hcat, dinv, w2):
    g = N // BN
    return pl.pallas_call(
        _pass2_body,
        grid=(g,),
        in_specs=[
            pl.BlockSpec((BN, N), lambda i: (i, 0)),
            pl.BlockSpec((N, 2 * H), lambda i: (0, 0)),
            pl.BlockSpec((BN, 1), lambda i: (i, 0)),
            pl.BlockSpec((H, H), lambda i: (0, 0)),
        ],
        out_specs=pl.BlockSpec((BN, 2 * H), lambda i: (i, 0)),
        out_shape=jax.ShapeDtypeStruct((N, 2 * H), jnp.bfloat16),
    )(adj, hcat, dinv, w2)


def _pass3_body(adj_ref, h_ref, dinv_ref, watt_ref, aatt_ref,
                pos_ref, neg_ref, whp_ref, whn_ref,
                f1p_ref, f2p_ref, f1n_ref, f2n_ref):
    y = jnp.dot(adj_ref[...], h_ref[...], preferred_element_type=jnp.float32)
    dinv = dinv_ref[...]
    pos = y[:, :H] * dinv
    neg = y[:, H:] * dinv
    pos_ref[...] = pos
    neg_ref[...] = neg
    watt = watt_ref[...]
    a = aatt_ref[...]
    a1 = a[:NHID, :]
    a2 = a[NHID:, :]
    whp = jnp.dot(pos, watt, preferred_element_type=jnp.float32)
    whn = jnp.dot(neg, watt, preferred_element_type=jnp.float32)
    whp_ref[...] = whp
    whn_ref[...] = whn
    f1p_ref[...] = jnp.dot(whp, a1, preferred_element_type=jnp.float32)
    f2p_ref[...] = jnp.dot(whp, a2, preferred_element_type=jnp.float32)
    f1n_ref[...] = jnp.dot(whn, a1, preferred_element_type=jnp.float32)
    f2n_ref[...] = jnp.dot(whn, a2, preferred_element_type=jnp.float32)


def _pass3(adj, hcat, dinv, watt, aatt):
    g = N // BN
    vec = jax.ShapeDtypeStruct((N, 1), jnp.float32)
    return pl.pallas_call(
        _pass3_body,
        grid=(g,),
        in_specs=[
            pl.BlockSpec((BN, N), lambda i: (i, 0)),
            pl.BlockSpec((N, 2 * H), lambda i: (0, 0)),
            pl.BlockSpec((BN, 1), lambda i: (i, 0)),
            pl.BlockSpec((H, NHID), lambda i: (0, 0)),
            pl.BlockSpec((2 * NHID, 1), lambda i: (0, 0)),
        ],
        out_specs=[
            pl.BlockSpec((BN, H), lambda i: (i, 0)),
            pl.BlockSpec((BN, H), lambda i: (i, 0)),
            pl.BlockSpec((BN, NHID), lambda i: (i, 0)),
            pl.BlockSpec((BN, NHID), lambda i: (i, 0)),
            pl.BlockSpec((BN, 1), lambda i: (i, 0)),
            pl.BlockSpec((BN, 1), lambda i: (i, 0)),
            pl.BlockSpec((BN, 1), lambda i: (i, 0)),
            pl.BlockSpec((BN, 1), lambda i: (i, 0)),
        ],
        out_shape=[
            jax.ShapeDtypeStruct((N, H), jnp.float32),
            jax.ShapeDtypeStruct((N, H), jnp.float32),
            jax.ShapeDtypeStruct((N, NHID), jnp.float32),
            jax.ShapeDtypeStruct((N, NHID), jnp.float32),
            vec, vec, vec, vec,
        ],
    )(adj, hcat, dinv, watt, aatt)


def _masked_att_matmul(adj, f1, f2row, wh_ext):
    f2m = jnp.max(f2row, axis=1, keepdims=True)
    t = f1 + f2m
    m = jnp.where(t >= 0.0, t, 0.2 * t)
    a = f1 - m
    b = 0.2 * f1 - m
    c2 = 0.2 * f2row
    q = jnp.maximum(a + f2row, b + c2)
    p = adj * jnp.exp(q).astype(jnp.bfloat16)
    hp_ext = jnp.dot(p, wh_ext, preferred_element_type=jnp.float32)
    d = hp_ext.shape[1] - 1
    return hp_ext[:, :d] / hp_ext[:, d:]


def _elu(x):
    return jnp.where(x > 0.0, x, jnp.exp(x) - 1.0)


def _gat1_body(adj_ref, f1p_ref, f2p_ref, f1n_ref, f2n_ref, whp_ref, whn_ref,
               wout_ref, aout_ref,
               wh2p_ref, wh2n_ref, g1p_ref, g2p_ref, g1n_ref, g2n_ref):
    adj = adj_ref[...]
    ones = jnp.ones((N, 1), dtype=jnp.float32)
    whpe = jnp.concatenate([whp_ref[...], ones], axis=1).astype(jnp.bfloat16)
    whne = jnp.concatenate([whn_ref[...], ones], axis=1).astype(jnp.bfloat16)
    xp = _elu(_masked_att_matmul(adj, f1p_ref[...], f2p_ref[...], whpe))
    xn = _elu(_masked_att_matmul(adj, f1n_ref[...], f2n_ref[...], whne))
    wout = wout_ref[...]
    a = aout_ref[...]
    a1 = a[:NCLASS, :]
    a2 = a[NCLASS:, :]
    wh2p = jnp.dot(xp, wout, preferred_element_type=jnp.float32)
    wh2n = jnp.dot(xn, wout, preferred_element_type=jnp.float32)
    wh2p_ref[...] = wh2p
    wh2n_ref[...] = wh2n
    g1p_ref[...] = jnp.dot(wh2p, a1, preferred_element_type=jnp.float32)
    g2p_ref[...] = jnp.dot(wh2p, a2, preferred_element_type=jnp.float32)
    g1n_ref[...] = jnp.dot(wh2n, a1, preferred_element_type=jnp.float32)
    g2n_ref[...] = jnp.dot(wh2n, a2, preferred_element_type=jnp.float32)


def _gat1(adj, f1p, f2p_row, f1n, f2n_row, whp, whn, wout, aout):
    g = N // BG
    vec = jax.ShapeDtypeStruct((N, 1), jnp.float32)
    blk = lambda r, c: pl.BlockSpec((r, c), lambda i: (i, 0))
    full = lambda r, c: pl.BlockSpec((r, c), lambda i: (0, 0))
    return pl.pallas_call(
        _gat1_body,
        grid=(g,),
        in_specs=[
            blk(BG, N),
            blk(BG, 1), full(1, N),
            blk(BG, 1), full(1, N),
            full(N, NHID), full(N, NHID),
            full(NHID, NCLASS), full(2 * NCLASS, 1),
        ],
        out_specs=[
            blk(BG, NCLASS), blk(BG, NCLASS),
            blk(BG, 1), blk(BG, 1), blk(BG, 1), blk(BG, 1),
        ],
        out_shape=[
            jax.ShapeDtypeStruct((N, NCLASS), jnp.float32),
            jax.ShapeDtypeStruct((N, NCLASS), jnp.float32),
            vec, vec, vec, vec,
        ],
    )(adj, f1p, f2p_row, f1n, f2n_row, whp, whn, wout, aout)


def _gat2_body(adj_ref, g1p_ref, g2p_ref, g1n_ref, g2n_ref, wh2p_ref, wh2n_ref,
               attp_ref, attn_ref):
    adj = adj_ref[...]
    ones = jnp.ones((N, 1), dtype=jnp.float32)
    wh2pe = jnp.concatenate([wh2p_ref[...], ones], axis=1).astype(jnp.bfloat16)
    wh2ne = jnp.concatenate([wh2n_ref[...], ones], axis=1).astype(jnp.bfloat16)
    attp_ref[...] = _elu(_masked_att_matmul(adj, g1p_ref[...], g2p_ref[...], wh2pe))
    attn_ref[...] = _elu(_masked_att_matmul(adj, g1n_ref[...], g2n_ref[...], wh2ne))


def _gat2(adj, g1p, g2p_row, g1n, g2n_row, wh2p, wh2n):
    g = N // BG
    blk = lambda r, c: pl.BlockSpec((r, c), lambda i: (i, 0))
    full = lambda r, c: pl.BlockSpec((r, c), lambda i: (0, 0))
    return pl.pallas_call(
        _gat2_body,
        grid=(g,),
        in_specs=[
            blk(BG, N),
            blk(BG, 1), full(1, N),
            blk(BG, 1), full(1, N),
            full(N, NCLASS), full(N, NCLASS),
        ],
        out_specs=[blk(BG, NCLASS), blk(BG, NCLASS)],
        out_shape=[
            jax.ShapeDtypeStruct((N, NCLASS), jnp.float32),
            jax.ShapeDtypeStruct((N, NCLASS), jnp.float32),
        ],
    )(adj, g1p, g2p_row, g1n, g2n_row, wh2p, wh2n)


def _cluster_body(pos_ref, mu_ref, mu_out_ref, r_out_ref, colmean_ref):
    pos = pos_ref[...]
    nrm = jnp.sqrt(jnp.sum(pos * pos, axis=1, keepdims=True))
    data = pos / (nrm + 1e-8)

    def norm_rows(m):
        return m / jnp.sqrt(jnp.sum(m * m, axis=1, keepdims=True))

    ones_col = jnp.ones((N, 1), dtype=jnp.bfloat16)
    data_bf = data.astype(jnp.bfloat16)

    def step(carry):
        mu, _ = carry
        mun = norm_rows(mu)
        dist = lax.dot_general(data_bf, mun.astype(jnp.bfloat16),
                               (((1,), (1,)), ((), ())),
                               preferred_element_type=jnp.float32)
        z = BETA * dist
        z = z - jnp.max(z, axis=1, keepdims=True)
        e = jnp.exp(z)
        r = (e / jnp.sum(e, axis=1, keepdims=True)).astype(jnp.bfloat16)
        cm = lax.dot_general(r, data_bf, (((0,), (0,)), ((), ())),
                             preferred_element_type=jnp.float32)
        cr = lax.dot_general(r, ones_col, (((0,), (0,)), ((), ())),
                             preferred_element_type=jnp.float32)
        return cm / cr, dist

    mu0 = mu_ref[...]
    mu, dist = lax.fori_loop(0, 11, lambda t, c: step(c),
                             (mu0, jnp.zeros((N, K), dtype=jnp.float32)))
    z = BETA * dist
    z = z - jnp.max(z, axis=1, keepdims=True)
    e = jnp.exp(z)
    r = e / jnp.sum(e, axis=1, keepdims=True)
    mu_out_ref[...] = mu
    r_out_ref[...] = r
    colmean_ref[...] = jnp.mean(pos, axis=0, keepdims=True)


def _cluster(pos, mu_init):
    return pl.pallas_call(
        _cluster_body,
        out_shape=[
            jax.ShapeDtypeStruct((K, H), jnp.float32),
            jax.ShapeDtypeStruct((N, K), jnp.float32),
            jax.ShapeDtypeStruct((1, H), jnp.float32),
        ],
    )(pos, mu_init)


def _bce_sum(x, z):
    return jnp.sum(jnp.maximum(x, 0.0) - x * z + jnp.log(1.0 + jnp.exp(-jnp.abs(x))))


def _finalize_body(pos_ref, neg_ref, r_ref, mu_ref, wdisc_ref, colmean_ref,
                   attp_ref, attn_ref, out_ref):
    i = pl.program_id(0)

    pos = pos_ref[...]
    neg = neg_ref[...]
    gs = 1.0 / (1.0 + jnp.exp(-colmean_ref[...]))
    v = lax.dot_general(wdisc_ref[...], gs, (((1,), (1,)), ((), ())),
                        preferred_element_type=jnp.float32)
    pos_graph = jnp.dot(pos, v, preferred_element_type=jnp.float32)
    neg_graph = jnp.dot(neg, v, preferred_element_type=jnp.float32)
    cs_logit = jnp.dot(r_ref[...].astype(jnp.bfloat16), mu_ref[...].astype(jnp.bfloat16),
                       preferred_element_type=jnp.float32)
    cs = 1.0 / (1.0 + jnp.exp(-cs_logit))
    pos_cluster = jnp.sum(pos * cs, axis=1, keepdims=True)
    neg_cluster = jnp.sum(neg * cs, axis=1, keepdims=True)

    part = ALPHA * (_bce_sum(pos_graph, 1.0) + _bce_sum(neg_graph, 0.0)) / N
    part += (1.0 - ALPHA) * (_bce_sum(pos_cluster, 1.0) + _bce_sum(neg_cluster, 0.0)) / N
    part += GAMMA * (_bce_sum(attp_ref[...], 1.0) + _bce_sum(attn_ref[...], 0.0)) / (N * NCLASS)

    @pl.when(i == 0)
    def _():
        out_ref[...] = jnp.zeros_like(out_ref)

    out_ref[...] = out_ref[...] + part


def _finalize(pos, neg, r, mu, wdisc, colmean, attp, attn):
    g = N // BN
    blk = lambda r_, c: pl.BlockSpec((r_, c), lambda i: (i, 0))
    full = lambda r_, c: pl.BlockSpec((r_, c), lambda i: (0, 0))
    return pl.pallas_call(
        _finalize_body,
        grid=(g,),
        in_specs=[
            blk(BN, H), blk(BN, H), blk(BN, K),
            full(K, H), full(H, H), full(1, H),
            blk(BN, NCLASS), blk(BN, NCLASS),
        ],
        out_specs=full(1, 1),
        out_shape=jax.ShapeDtypeStruct((1, 1), jnp.float32),
    )(pos, neg, r, mu, wdisc, colmean, attp, attn)


def kernel(features, adj, perm, W_gcn1, W_gcn2, W_disc, W_att, a_att, W_out, a_out, mu_init):
    featp = _gather_rows(features, perm.astype(jnp.int32))
    hcat0, dinv, adj_c = _pass1(adj, features, featp, W_gcn1)
    hcat1 = _pass2(adj_c, hcat0, dinv, W_gcn2)
    (pos, neg, whp, whn, f1p, f2p, f1n, f2n) = _pass3(adj_c, hcat1, dinv, W_att, a_att)
    (wh2p, wh2n, g1p, g2p, g1n, g2n) = _gat1(
        adj_c, f1p, f2p.reshape(1, N), f1n, f2n.reshape(1, N), whp, whn, W_out, a_out)
    attp, attn = _gat2(
        adj_c, g1p, g2p.reshape(1, N), g1n, g2n.reshape(1, N), wh2p, wh2n)
    mu, r, colmean = _cluster(pos, mu_init)
    lmat = _finalize(pos, neg, r, mu, W_disc, colmean, attp, attn)
    return lmat[0, 0]


# speedup vs baseline: 1.3527x; 1.0526x over previous
import functools

import jax
import jax.numpy as jnp
from jax import lax
from jax.experimental import pallas as pl
from jax.experimental.pallas import tpu as pltpu
from jax.experimental.pallas import tpu_sc as plsc

N = 4096
F = 512
H = 256
K = 128
NHID = 8
NCLASS = 2
BETA = 100.0
ALPHA = 0.5
GAMMA = 0.5

BN = 256
BG = 256
NEG_BIG = -9e15


def _f32(x):
    return x.astype(jnp.float32)


@functools.cache
def _make_sc_gather():
    nc, ns = 2, 16
    nw = nc * ns
    bpw = N // nw
    mesh = plsc.VectorSubcoreMesh(core_axis_name="c", subcore_axis_name="s")

    @functools.partial(
        pl.kernel,
        mesh=mesh,
        out_type=jax.ShapeDtypeStruct((N, F), jnp.float32),
        scratch_types=[
            pltpu.VMEM((bpw,), jnp.int32),
            pltpu.VMEM((bpw, F), jnp.float32),
            pltpu.SemaphoreType.DMA,
        ],
    )
    def gather_k(table_hbm, idx_hbm, out_hbm, idx_v, rows_v, sem):
        wid = lax.axis_index("s") * nc + lax.axis_index("c")
        base = wid * bpw
        pltpu.sync_copy(idx_hbm.at[pl.ds(base, bpw)], idx_v)
        pltpu.async_copy(table_hbm.at[idx_v], rows_v, sem).wait()
        pltpu.sync_copy(rows_v, out_hbm.at[pl.ds(base, bpw)])

    return gather_k


def _gather_rows(table, idx):
    return _make_sc_gather()(table, idx)


def _pass1_body(adj_ref, x_ref, xp_ref, w1_ref, hcat_ref, dinv_ref, adjc_ref):
    a = adj_ref[...]
    adjc_ref[...] = a.astype(jnp.bfloat16)
    deg = jnp.sum(a, axis=1, keepdims=True)
    dinv = lax.rsqrt(deg)
    w1 = w1_ref[...]
    xw = jnp.dot(x_ref[...], w1, preferred_element_type=jnp.float32)
    xwn = jnp.dot(xp_ref[...], w1, preferred_element_type=jnp.float32)
    hcat_ref[...] = (jnp.concatenate([xw, xwn], axis=1) * dinv).astype(jnp.bfloat16)
    dinv_ref[...] = dinv


def _pass1(adj, x, xp, w1):
    g = N // BN
    return pl.pallas_call(
        _pass1_body,
        grid=(g,),
        in_specs=[
            pl.BlockSpec((BN, N), lambda i: (i, 0)),
            pl.BlockSpec((BN, F), lambda i: (i, 0)),
            pl.BlockSpec((BN, F), lambda i: (i, 0)),
            pl.BlockSpec((F, H), lambda i: (0, 0)),
        ],
        out_specs=[
            pl.BlockSpec((BN, 2 * H), lambda i: (i, 0)),
            pl.BlockSpec((BN, 1), lambda i: (i, 0)),
            pl.BlockSpec((BN, N), lambda i: (i, 0)),
        ],
        out_shape=[
            jax.ShapeDtypeStruct((N, 2 * H), jnp.bfloat16),
            jax.ShapeDtypeStruct((N, 1), jnp.float32),
            jax.ShapeDtypeStruct((N, N), jnp.bfloat16),
        ],
    )(adj, x, xp, w1)


def _pass2_body(adj_ref, h_ref, dinv_ref, w2_ref, out_ref):
    y = jnp.dot(adj_ref[...], h_ref[...], preferred_element_type=jnp.float32)
    h = jnp.maximum(y * dinv_ref[...], 0.0).astype(jnp.bfloat16)
    w2 = w2_ref[...].astype(jnp.bfloat16)
    zp = jnp.dot(h[:, :H], w2, preferred_element_type=jnp.float32)
    zn = jnp.dot(h[:, H:], w2, preferred_element_type=jnp.float32)
    out_ref[...] = (jnp.concatenate([zp, zn], axis=1) * dinv_ref[...]).astype(jnp.bfloat16)


def _pass2(adj, hcat, dinv, w2):
    g = N // BN
    return pl.pallas_call(
        _pass2_body,
        grid=(g,),
        in_specs=[
            pl.BlockSpec((BN, N), lambda i: (i, 0)),
            pl.BlockSpec((N, 2 * H), lambda i: (0, 0)),
            pl.BlockSpec((BN, 1), lambda i: (i, 0)),
            pl.BlockSpec((H, H), lambda i: (0, 0)),
        ],
        out_specs=pl.BlockSpec((BN, 2 * H), lambda i: (i, 0)),
        out_shape=jax.ShapeDtypeStruct((N, 2 * H), jnp.bfloat16),
    )(adj, hcat, dinv, w2)


def _pass3_body(adj_ref, h_ref, dinv_ref, watt_ref, aatt_ref,
                pos_ref, neg_ref, whp_ref, whn_ref,
                f1p_ref, f2p_ref, f1n_ref, f2n_ref, posT_ref, rnorm_ref):
    y = jnp.dot(adj_ref[...], h_ref[...], preferred_element_type=jnp.float32)
    dinv = dinv_ref[...]
    pos = y[:, :H] * dinv
    neg = y[:, H:] * dinv
    pos_ref[...] = pos
    neg_ref[...] = neg
    posT_ref[...] = pos.T.astype(jnp.bfloat16)
    rnorm_ref[...] = jnp.sqrt(jnp.sum(pos * pos, axis=1, keepdims=True))
    watt = watt_ref[...]
    a = aatt_ref[...]
    a1 = a[:NHID, :]
    a2 = a[NHID:, :]
    whp = jnp.dot(pos, watt, preferred_element_type=jnp.float32)
    whn = jnp.dot(neg, watt, preferred_element_type=jnp.float32)
    whp_ref[...] = whp
    whn_ref[...] = whn
    f1p_ref[...] = jnp.dot(whp, a1, preferred_element_type=jnp.float32)
    f2p_ref[...] = jnp.dot(whp, a2, preferred_element_type=jnp.float32)
    f1n_ref[...] = jnp.dot(whn, a1, preferred_element_type=jnp.float32)
    f2n_ref[...] = jnp.dot(whn, a2, preferred_element_type=jnp.float32)


def _pass3(adj, hcat, dinv, watt, aatt):
    g = N // BN
    vec = jax.ShapeDtypeStruct((N, 1), jnp.float32)
    return pl.pallas_call(
        _pass3_body,
        grid=(g,),
        in_specs=[
            pl.BlockSpec((BN, N), lambda i: (i, 0)),
            pl.BlockSpec((N, 2 * H), lambda i: (0, 0)),
            pl.BlockSpec((BN, 1), lambda i: (i, 0)),
            pl.BlockSpec((H, NHID), lambda i: (0, 0)),
            pl.BlockSpec((2 * NHID, 1), lambda i: (0, 0)),
        ],
        out_specs=[
            pl.BlockSpec((BN, H), lambda i: (i, 0)),
            pl.BlockSpec((BN, H), lambda i: (i, 0)),
            pl.BlockSpec((BN, NHID), lambda i: (i, 0)),
            pl.BlockSpec((BN, NHID), lambda i: (i, 0)),
            pl.BlockSpec((BN, 1), lambda i: (i, 0)),
            pl.BlockSpec((BN, 1), lambda i: (i, 0)),
            pl.BlockSpec((BN, 1), lambda i: (i, 0)),
            pl.BlockSpec((BN, 1), lambda i: (i, 0)),
            pl.BlockSpec((H, BN), lambda i: (0, i)),
            pl.BlockSpec((BN, 1), lambda i: (i, 0)),
        ],
        out_shape=[
            jax.ShapeDtypeStruct((N, H), jnp.float32),
            jax.ShapeDtypeStruct((N, H), jnp.float32),
            jax.ShapeDtypeStruct((N, NHID), jnp.float32),
            jax.ShapeDtypeStruct((N, NHID), jnp.float32),
            vec, vec, vec, vec,
            jax.ShapeDtypeStruct((H, N), jnp.bfloat16),
            jax.ShapeDtypeStruct((N, 1), jnp.float32),
        ],
    )(adj, hcat, dinv, watt, aatt)


def _masked_att_matmul(adj, f1, f2row, wh_ext):
    f2m = jnp.max(f2row, axis=1, keepdims=True)
    t = f1 + f2m
    m = jnp.where(t >= 0.0, t, 0.2 * t)
    a = f1 - m
    b = 0.2 * f1 - m
    c2 = 0.2 * f2row
    q = jnp.maximum(a + f2row, b + c2)
    p = adj * jnp.exp(q).astype(jnp.bfloat16)
    hp_ext = jnp.dot(p, wh_ext, preferred_element_type=jnp.float32)
    d = hp_ext.shape[1] - 1
    return hp_ext[:, :d] / hp_ext[:, d:]


def _elu(x):
    return jnp.where(x > 0.0, x, jnp.exp(x) - 1.0)


def _gat1_body(adj_ref, f1p_ref, f2p_ref, f1n_ref, f2n_ref, whp_ref, whn_ref,
               wout_ref, aout_ref,
               wh2p_ref, wh2n_ref, g1p_ref, g2p_ref, g1n_ref, g2n_ref):
    adj = adj_ref[...]
    ones = jnp.ones((N, 1), dtype=jnp.float32)
    whpe = jnp.concatenate([whp_ref[...], ones], axis=1).astype(jnp.bfloat16)
    whne = jnp.concatenate([whn_ref[...], ones], axis=1).astype(jnp.bfloat16)
    xp = _elu(_masked_att_matmul(adj, f1p_ref[...], f2p_ref[...], whpe))
    xn = _elu(_masked_att_matmul(adj, f1n_ref[...], f2n_ref[...], whne))
    wout = wout_ref[...]
    a = aout_ref[...]
    a1 = a[:NCLASS, :]
    a2 = a[NCLASS:, :]
    wh2p = jnp.dot(xp, wout, preferred_element_type=jnp.float32)
    wh2n = jnp.dot(xn, wout, preferred_element_type=jnp.float32)
    wh2p_ref[...] = wh2p
    wh2n_ref[...] = wh2n
    g1p_ref[...] = jnp.dot(wh2p, a1, preferred_element_type=jnp.float32)
    g2p_ref[...] = jnp.dot(wh2p, a2, preferred_element_type=jnp.float32)
    g1n_ref[...] = jnp.dot(wh2n, a1, preferred_element_type=jnp.float32)
    g2n_ref[...] = jnp.dot(wh2n, a2, preferred_element_type=jnp.float32)


def _gat1(adj, f1p, f2p_row, f1n, f2n_row, whp, whn, wout, aout):
    g = N // BG
    vec = jax.ShapeDtypeStruct((N, 1), jnp.float32)
    blk = lambda r, c: pl.BlockSpec((r, c), lambda i: (i, 0))
    full = lambda r, c: pl.BlockSpec((r, c), lambda i: (0, 0))
    return pl.pallas_call(
        _gat1_body,
        grid=(g,),
        in_specs=[
            blk(BG, N),
            blk(BG, 1), full(1, N),
            blk(BG, 1), full(1, N),
            full(N, NHID), full(N, NHID),
            full(NHID, NCLASS), full(2 * NCLASS, 1),
        ],
        out_specs=[
            blk(BG, NCLASS), blk(BG, NCLASS),
            blk(BG, 1), blk(BG, 1), blk(BG, 1), blk(BG, 1),
        ],
        out_shape=[
            jax.ShapeDtypeStruct((N, NCLASS), jnp.float32),
            jax.ShapeDtypeStruct((N, NCLASS), jnp.float32),
            vec, vec, vec, vec,
        ],
    )(adj, f1p, f2p_row, f1n, f2n_row, whp, whn, wout, aout)


def _gat2_body(adj_ref, g1p_ref, g2p_ref, g1n_ref, g2n_ref, wh2p_ref, wh2n_ref,
               attp_ref, attn_ref):
    adj = adj_ref[...]
    ones = jnp.ones((N, 1), dtype=jnp.float32)
    wh2pe = jnp.concatenate([wh2p_ref[...], ones], axis=1).astype(jnp.bfloat16)
    wh2ne = jnp.concatenate([wh2n_ref[...], ones], axis=1).astype(jnp.bfloat16)
    attp_ref[...] = _elu(_masked_att_matmul(adj, g1p_ref[...], g2p_ref[...], wh2pe))
    attn_ref[...] = _elu(_masked_att_matmul(adj, g1n_ref[...], g2n_ref[...], wh2ne))


def _gat2(adj, g1p, g2p_row, g1n, g2n_row, wh2p, wh2n):
    g = N // BG
    blk = lambda r, c: pl.BlockSpec((r, c), lambda i: (i, 0))
    full = lambda r, c: pl.BlockSpec((r, c), lambda i: (0, 0))
    return pl.pallas_call(
        _gat2_body,
        grid=(g,),
        in_specs=[
            blk(BG, N),
            blk(BG, 1), full(1, N),
            blk(BG, 1), full(1, N),
            full(N, NCLASS), full(N, NCLASS),
        ],
        out_specs=[blk(BG, NCLASS), blk(BG, NCLASS)],
        out_shape=[
            jax.ShapeDtypeStruct((N, NCLASS), jnp.float32),
            jax.ShapeDtypeStruct((N, NCLASS), jnp.float32),
        ],
    )(adj, g1p, g2p_row, g1n, g2n_row, wh2p, wh2n)


def _cluster_body(posT_ref, rnT_ref, mu_ref, mu_out_ref, rT_out_ref, colmeanT_ref):
    posT = posT_ref[...].astype(jnp.float32)
    invn = 1.0 / (rnT_ref[...] + 1e-8)
    dataT_bf = (posT * invn).astype(jnp.bfloat16)

    def norm_rows(m):
        return m / jnp.sqrt(jnp.sum(m * m, axis=1, keepdims=True))

    def step(carry):
        mu, _ = carry
        mun = norm_rows(mu)
        distT = lax.dot_general(mun.astype(jnp.bfloat16), dataT_bf,
                                (((1,), (0,)), ((), ())),
                                preferred_element_type=jnp.float32)
        z = BETA * distT
        z = z - jnp.max(z, axis=0, keepdims=True)
        e = jnp.exp(z)
        rT = e / jnp.sum(e, axis=0, keepdims=True)
        cm = lax.dot_general(rT.astype(jnp.bfloat16), dataT_bf,
                             (((1,), (1,)), ((), ())),
                             preferred_element_type=jnp.float32)
        cr = jnp.sum(rT, axis=1, keepdims=True)
        return cm / cr, distT

    mu0 = mu_ref[...]
    mu, distT = lax.fori_loop(0, 11, lambda t, c: step(c),
                              (mu0, jnp.zeros((K, N), dtype=jnp.float32)))
    z = BETA * distT
    z = z - jnp.max(z, axis=0, keepdims=True)
    e = jnp.exp(z)
    rT = e / jnp.sum(e, axis=0, keepdims=True)
    mu_out_ref[...] = mu
    rT_out_ref[...] = rT
    colmeanT_ref[...] = jnp.mean(posT, axis=1, keepdims=True)


def _cluster(posT, rnT, mu_init):
    return pl.pallas_call(
        _cluster_body,
        out_shape=[
            jax.ShapeDtypeStruct((K, H), jnp.float32),
            jax.ShapeDtypeStruct((K, N), jnp.float32),
            jax.ShapeDtypeStruct((H, 1), jnp.float32),
        ],
    )(posT, rnT, mu_init)


def _bce_sum(x, z):
    return jnp.sum(jnp.maximum(x, 0.0) - x * z + jnp.log(1.0 + jnp.exp(-jnp.abs(x))))


def _finalize_body(pos_ref, neg_ref, rT_ref, mu_ref, wdisc_ref, colmeanT_ref,
                   attp_ref, attn_ref, out_ref):
    i = pl.program_id(0)

    pos = pos_ref[...]
    neg = neg_ref[...]
    gs = 1.0 / (1.0 + jnp.exp(-colmeanT_ref[...]))
    v = jnp.dot(wdisc_ref[...], gs, preferred_element_type=jnp.float32)
    pos_graph = jnp.dot(pos, v, preferred_element_type=jnp.float32)
    neg_graph = jnp.dot(neg, v, preferred_element_type=jnp.float32)
    cs_logit = lax.dot_general(rT_ref[...].astype(jnp.bfloat16),
                               mu_ref[...].astype(jnp.bfloat16),
                               (((0,), (0,)), ((), ())),
                               preferred_element_type=jnp.float32)
    cs = 1.0 / (1.0 + jnp.exp(-cs_logit))
    pos_cluster = jnp.sum(pos * cs, axis=1, keepdims=True)
    neg_cluster = jnp.sum(neg * cs, axis=1, keepdims=True)

    part = ALPHA * (_bce_sum(pos_graph, 1.0) + _bce_sum(neg_graph, 0.0)) / N
    part += (1.0 - ALPHA) * (_bce_sum(pos_cluster, 1.0) + _bce_sum(neg_cluster, 0.0)) / N
    part += GAMMA * (_bce_sum(attp_ref[...], 1.0) + _bce_sum(attn_ref[...], 0.0)) / (N * NCLASS)

    @pl.when(i == 0)
    def _():
        out_ref[...] = jnp.zeros_like(out_ref)

    out_ref[...] = out_ref[...] + part


def _finalize(pos, neg, rT, mu, wdisc, colmeanT, attp, attn):
    g = N // BN
    blk = lambda r_, c: pl.BlockSpec((r_, c), lambda i: (i, 0))
    full = lambda r_, c: pl.BlockSpec((r_, c), lambda i: (0, 0))
    return pl.pallas_call(
        _finalize_body,
        grid=(g,),
        in_specs=[
            blk(BN, H), blk(BN, H),
            pl.BlockSpec((K, BN), lambda i: (0, i)),
            full(K, H), full(H, H), full(H, 1),
            blk(BN, NCLASS), blk(BN, NCLASS),
        ],
        out_specs=full(1, 1),
        out_shape=jax.ShapeDtypeStruct((1, 1), jnp.float32),
    )(pos, neg, rT, mu, wdisc, colmeanT, attp, attn)


def kernel(features, adj, perm, W_gcn1, W_gcn2, W_disc, W_att, a_att, W_out, a_out, mu_init):
    featp = _gather_rows(features, perm.astype(jnp.int32))
    hcat0, dinv, adj_c = _pass1(adj, features, featp, W_gcn1)
    hcat1 = _pass2(adj_c, hcat0, dinv, W_gcn2)
    (pos, neg, whp, whn, f1p, f2p, f1n, f2n, posT, rnorm) = _pass3(
        adj_c, hcat1, dinv, W_att, a_att)
    (wh2p, wh2n, g1p, g2p, g1n, g2n) = _gat1(
        adj_c, f1p, f2p.reshape(1, N), f1n, f2n.reshape(1, N), whp, whn, W_out, a_out)
    attp, attn = _gat2(
        adj_c, g1p, g2p.reshape(1, N), g1n, g2n.reshape(1, N), wh2p, wh2n)
    mu, rT, colmeanT = _cluster(posT, rnorm.reshape(1, N), mu_init)
    lmat = _finalize(pos, neg, rT, mu, W_disc, colmeanT, attp, attn)
    return lmat[0, 0]
